# baseline (speedup 1.0000x reference)
import functools

import jax
import jax.numpy as jnp
import numpy as np
from jax import lax
from jax.experimental import pallas as pl
from jax.experimental.pallas import tpu as pltpu
from jax.experimental.pallas import tpu_sc as plsc

_VOCAB = 100000
_EMBED = 64
_NUM_CLASS = 4
_BATCH = 4096
_CUTLEN = 200

_NC = 2
_NS = 16
_NW = _NC * _NS
_BPW = _BATCH // _NW
_TPW = _BPW * _CUTLEN
_G = 64
_NGROUPS = _TPW // _G

_NBUF = 8
_AHEAD = 6

_DIV200_MUL = 5243
_DIV200_SHIFT = 20


def _sc_body(text_ref, table_ref, out_ref, idx_v, seg_v, rows_v,
             acc_sh, sem_g, sem_s):
  c = lax.axis_index("c")
  s = lax.axis_index("s")
  wid = c * _NS + s

  pltpu.sync_copy(text_ref.at[pl.ds(wid * _TPW, _TPW)], idx_v)

  def _zero_row(r, carry):
    for q in range(_EMBED // 16):
      rows_v[0, r, pl.ds(q * 16, 16)] = jnp.zeros((16,), jnp.float32)
    return carry

  lax.fori_loop(0, _G, _zero_row, 0)
  for z in range(_BPW // _G):
    pltpu.sync_copy(rows_v.at[0], acc_sh.at[pl.ds(s * _BPW + z * _G, _G)])

  lanes = lax.iota(jnp.int32, 16)
  base = s * _BPW

  def _gen_seg(j, carry):
    for q in range(_G // 16):
      p = lanes + (j * _G + q * 16)
      seg_v[j, pl.ds(q * 16, 16)] = (
          base + ((p * _DIV200_MUL) >> _DIV200_SHIFT))
    return carry

  lax.fori_loop(0, _NGROUPS, _gen_seg, 0)

  for j in range(_AHEAD):
    pltpu.async_copy(table_ref.at[idx_v.at[pl.ds(j * _G, _G)]],
                     rows_v.at[j], sem_g)

  def _group(j, carry):
    slot = lax.rem(j, _NBUF)
    pltpu.make_async_copy(table_ref.at[idx_v.at[pl.ds(j * _G, _G)]],
                          rows_v.at[slot], sem_g).wait()
    pltpu.async_copy(rows_v.at[slot], acc_sh.at[seg_v.at[j]], sem_s,
                     add=True)

    @pl.when(j >= 1)
    def _drain_prev():
      pslot = lax.rem(j - 1, _NBUF)
      pltpu.make_async_copy(rows_v.at[pslot], acc_sh.at[seg_v.at[j - 1]],
                            sem_s).wait()

    @pl.when(j + _AHEAD < _NGROUPS)
    def _fire_next():
      nslot = lax.rem(j + _AHEAD, _NBUF)
      pltpu.async_copy(table_ref.at[idx_v.at[pl.ds((j + _AHEAD) * _G, _G)]],
                       rows_v.at[nslot], sem_g)

    return carry

  lax.fori_loop(0, _NGROUPS, _group, 0)

  last = _NGROUPS - 1
  pltpu.make_async_copy(rows_v.at[last % _NBUF], acc_sh.at[seg_v.at[last]],
                        sem_s).wait()

  pltpu.sync_copy(acc_sh.at[pl.ds(s * _BPW, _BPW)],
                  out_ref.at[pl.ds(wid * _BPW, _BPW)])


@jax.jit
def _segment_sums(text, table):
  mesh = plsc.VectorSubcoreMesh(core_axis_name="c", subcore_axis_name="s",
                                num_cores=_NC, num_subcores=_NS)
  fn = pl.kernel(
      _sc_body,
      out_type=jax.ShapeDtypeStruct((_BATCH, _EMBED), jnp.float32),
      mesh=mesh,
      scratch_types=[
          pltpu.VMEM((_TPW,), jnp.int32),
          pltpu.VMEM((_NGROUPS, _G), jnp.int32),
          pltpu.VMEM((_NBUF, _G, _EMBED), jnp.float32),
          pltpu.VMEM_SHARED((_NS * _BPW, _EMBED), jnp.float32),
          pltpu.SemaphoreType.DMA,
          pltpu.SemaphoreType.DMA,
      ],
      compiler_params=pltpu.CompilerParams(use_tc_tiling_on_sc=False),
  )
  return fn(text, table)


def _tc_body(p_ref, w_ref, b_ref, o_ref):
  logits = jnp.dot(p_ref[...], w_ref[...],
                   preferred_element_type=jnp.float32) + b_ref[...]
  m = jnp.max(logits, axis=1, keepdims=True)
  e = jnp.exp(logits - m)
  o_ref[...] = e / jnp.sum(e, axis=1, keepdims=True)


@jax.jit
def _dense_softmax(pooled_sum, wt, b2):
  return pl.pallas_call(
      _tc_body,
      out_shape=jax.ShapeDtypeStruct((_BATCH, _NUM_CLASS), jnp.float32),
  )(pooled_sum, wt, b2)


def kernel(text, table, W, b):
  wt = (W.astype(jnp.float32) * (1.0 / _CUTLEN)).T
  b2 = b.reshape(1, _NUM_CLASS).astype(jnp.float32)

  pooled_sum = _segment_sums(text, table)
  return _dense_softmax(pooled_sum, wt, b2)


# speedup vs baseline: 1.3661x; 1.3661x over previous
import functools

import jax
import jax.numpy as jnp
import numpy as np
from jax import lax
from jax.experimental import pallas as pl
from jax.experimental.pallas import tpu as pltpu
from jax.experimental.pallas import tpu_sc as plsc

_VOCAB = 100000
_EMBED = 64
_NUM_CLASS = 4
_BATCH = 4096
_CUTLEN = 200

_NC = 2
_NS = 16
_NW = _NC * _NS
_BPW = _BATCH // _NW
_TPW = _BPW * _CUTLEN
_G = 128
_NGROUPS = _TPW // _G
_NPAIR = _NGROUPS // 2

_NBUF = 8
_AHEAD = 6

_DIV200_MUL = 5243
_DIV200_SHIFT = 20

_Q = _EMBED // 16


def _sc_body(text_ref, table_ref, out_ref, idx_v, seg_v, rows_v, acc_v,
             acc_sh, sem_g, sem_s):
  c = lax.axis_index("c")
  s = lax.axis_index("s")
  wid = c * _NS + s

  pltpu.sync_copy(text_ref.at[pl.ds(wid * _TPW, _TPW)], idx_v)

  zeros16 = jnp.zeros((16,), jnp.float32)

  def _zero_row(r, carry):
    for q in range(_Q):
      rows_v[0, r, pl.ds(q * 16, 16)] = zeros16
      acc_v[r, pl.ds(q * 16, 16)] = zeros16
    return carry

  lax.fori_loop(0, _BPW, _zero_row, 0)
  pltpu.sync_copy(rows_v.at[0], acc_sh.at[pl.ds(s * _BPW, _BPW)])

  lanes = lax.iota(jnp.int32, 16)
  base = s * _BPW

  def _gen_seg(jj, carry):
    for q in range(_G // 16):
      p = lanes + (jj * 2 * _G + q * 16)
      seg_v[jj, pl.ds(q * 16, 16)] = (
          base + ((p * _DIV200_MUL) >> _DIV200_SHIFT))
    return carry

  lax.fori_loop(0, _NPAIR, _gen_seg, 0)

  for j in range(_AHEAD):
    pltpu.async_copy(table_ref.at[idx_v.at[pl.ds(j * _G, _G)]],
                     rows_v.at[j], sem_g)

  def _gwait(j, slot):
    pltpu.make_async_copy(table_ref.at[idx_v.at[pl.ds(j * _G, _G)]],
                          rows_v.at[slot], sem_g).wait()

  def _pair(jj, carry):
    j0 = jj * 2
    j1 = j0 + 1
    slot0 = lax.rem(j0, _NBUF)
    slot1 = lax.rem(j1, _NBUF)

    _gwait(j0, slot0)
    pltpu.async_copy(rows_v.at[slot0], acc_sh.at[seg_v.at[jj]], sem_s,
                     add=True)

    @pl.when(jj >= 1)
    def _drain_prev():
      pslot = lax.rem(j0 - 2, _NBUF)
      pltpu.make_async_copy(rows_v.at[pslot], acc_sh.at[seg_v.at[jj - 1]],
                            sem_s).wait()

    @pl.when(j0 + _AHEAD < _NGROUPS)
    def _fire0():
      nslot = lax.rem(j0 + _AHEAD, _NBUF)
      pltpu.async_copy(table_ref.at[idx_v.at[pl.ds((j0 + _AHEAD) * _G, _G)]],
                       rows_v.at[nslot], sem_g)

    @pl.when(j1 + _AHEAD < _NGROUPS)
    def _fire1():
      nslot = lax.rem(j1 + _AHEAD, _NBUF)
      pltpu.async_copy(table_ref.at[idx_v.at[pl.ds((j1 + _AHEAD) * _G, _G)]],
                       rows_v.at[nslot], sem_g)

    _gwait(j1, slot1)
    p0 = j1 * _G
    b0 = (p0 * _DIV200_MUL) >> _DIV200_SHIFT
    bnd = jnp.minimum((b0 + 1) * _CUTLEN - p0, _G)

    def _accum(r, acc):
      return tuple(
          acc[q] + rows_v[slot1, r, pl.ds(q * 16, 16)] for q in range(_Q))

    z4 = (zeros16,) * _Q
    acc_a = lax.fori_loop(0, bnd, _accum, z4)
    acc_b = lax.fori_loop(bnd, _G, _accum, z4)
    for q in range(_Q):
      acc_v[b0, pl.ds(q * 16, 16)] = (
          acc_v[b0, pl.ds(q * 16, 16)] + acc_a[q])

    @pl.when(bnd < _G)
    def _flush_b():
      for q in range(_Q):
        acc_v[b0 + 1, pl.ds(q * 16, 16)] = (
            acc_v[b0 + 1, pl.ds(q * 16, 16)] + acc_b[q])

    return carry

  lax.fori_loop(0, _NPAIR, _pair, 0)

  pltpu.make_async_copy(rows_v.at[(_NGROUPS - 2) % _NBUF],
                        acc_sh.at[seg_v.at[_NPAIR - 1]], sem_s).wait()
  pltpu.sync_copy(acc_sh.at[pl.ds(s * _BPW, _BPW)], rows_v.at[0])

  def _merge_row(r, carry):
    for q in range(_Q):
      acc_v[r, pl.ds(q * 16, 16)] = (
          acc_v[r, pl.ds(q * 16, 16)] + rows_v[0, r, pl.ds(q * 16, 16)])
    return carry

  lax.fori_loop(0, _BPW, _merge_row, 0)
  pltpu.sync_copy(acc_v, out_ref.at[pl.ds(wid * _BPW, _BPW)])


@jax.jit
def _segment_sums(text, table):
  mesh = plsc.VectorSubcoreMesh(core_axis_name="c", subcore_axis_name="s",
                                num_cores=_NC, num_subcores=_NS)
  fn = pl.kernel(
      _sc_body,
      out_type=jax.ShapeDtypeStruct((_BATCH, _EMBED), jnp.float32),
      mesh=mesh,
      scratch_types=[
          pltpu.VMEM((_TPW,), jnp.int32),
          pltpu.VMEM((_NPAIR, _G), jnp.int32),
          pltpu.VMEM((_NBUF, _G, _EMBED), jnp.float32),
          pltpu.VMEM((_BPW, _EMBED), jnp.float32),
          pltpu.VMEM_SHARED((_NS * _BPW, _EMBED), jnp.float32),
          pltpu.SemaphoreType.DMA,
          pltpu.SemaphoreType.DMA,
      ],
      compiler_params=pltpu.CompilerParams(use_tc_tiling_on_sc=False),
  )
  return fn(text, table)


def _tc_body(p_ref, w_ref, b_ref, o_ref):
  logits = jnp.dot(p_ref[...], w_ref[...],
                   preferred_element_type=jnp.float32) + b_ref[...]
  m = jnp.max(logits, axis=1, keepdims=True)
  e = jnp.exp(logits - m)
  probs = e / jnp.sum(e, axis=1, keepdims=True)
  o_ref[...] = probs.T


@jax.jit
def _dense_softmax(pooled_sum, wt, b2):
  out_t = pl.pallas_call(
      _tc_body,
      out_shape=jax.ShapeDtypeStruct((_NUM_CLASS, _BATCH), jnp.float32),
  )(pooled_sum, wt, b2)
  return out_t.T


def kernel(text, table, W, b):
  wt = (W.astype(jnp.float32) * (1.0 / _CUTLEN)).T
  b2 = b.reshape(1, _NUM_CLASS).astype(jnp.float32)

  pooled_sum = _segment_sums(text, table)
  return _dense_softmax(pooled_sum, wt, b2)


# speedup vs baseline: 1.3821x; 1.0118x over previous
import functools

import jax
import jax.numpy as jnp
import numpy as np
from jax import lax
from jax.experimental import pallas as pl
from jax.experimental.pallas import tpu as pltpu
from jax.experimental.pallas import tpu_sc as plsc

_VOCAB = 100000
_EMBED = 64
_NUM_CLASS = 4
_BATCH = 4096
_CUTLEN = 200

_NC = 2
_NS = 16
_NW = _NC * _NS
_BPW = _BATCH // _NW
_TPW = _BPW * _CUTLEN
_G = 128
_NGROUPS = _TPW // _G
_BLK = 4
_NVEC = _BLK - 1
_NBLK = _NGROUPS // _BLK

_NBUF = 10
_AHEAD = 6

_DIV200_MUL = 5243
_DIV200_SHIFT = 20

_Q = _EMBED // 16


def _sc_body(text_ref, table_ref, out_ref, idx_v, seg_v, rows_v, acc_v,
             acc_sh, sem_g, sem_s):
  c = lax.axis_index("c")
  s = lax.axis_index("s")
  wid = c * _NS + s

  pltpu.sync_copy(text_ref.at[pl.ds(wid * _TPW, _TPW)], idx_v)

  zeros16 = jnp.zeros((16,), jnp.float32)

  def _zero_row(r, carry):
    for q in range(_Q):
      rows_v[0, r, pl.ds(q * 16, 16)] = zeros16
      acc_v[r, pl.ds(q * 16, 16)] = zeros16
    return carry

  lax.fori_loop(0, _BPW, _zero_row, 0)
  pltpu.sync_copy(rows_v.at[0], acc_sh.at[pl.ds(s * _BPW, _BPW)])

  lanes = lax.iota(jnp.int32, 16)
  base = s * _BPW

  def _gen_seg(jj, carry):
    for q in range(_G // 16):
      p = lanes + (jj * _BLK * _G + q * 16)
      seg_v[jj, pl.ds(q * 16, 16)] = (
          base + ((p * _DIV200_MUL) >> _DIV200_SHIFT))
    return carry

  lax.fori_loop(0, _NBLK, _gen_seg, 0)

  for j in range(_AHEAD):
    pltpu.async_copy(table_ref.at[idx_v.at[pl.ds(j * _G, _G)]],
                     rows_v.at[j], sem_g)

  def _gwait(j, slot):
    pltpu.make_async_copy(table_ref.at[idx_v.at[pl.ds(j * _G, _G)]],
                          rows_v.at[slot], sem_g).wait()

  def _block(jj, carry):
    j0 = jj * _BLK
    slot0 = lax.rem(j0, _NBUF)

    _gwait(j0, slot0)
    pltpu.async_copy(rows_v.at[slot0], acc_sh.at[seg_v.at[jj]], sem_s,
                     add=True)

    @pl.when(jj >= 1)
    def _drain_prev():
      pslot = lax.rem(j0 - _BLK, _NBUF)
      pltpu.make_async_copy(rows_v.at[pslot], acc_sh.at[seg_v.at[jj - 1]],
                            sem_s).wait()

    @pl.when(j0 + _AHEAD < _NGROUPS)
    def _fire0():
      nslot = lax.rem(j0 + _AHEAD, _NBUF)
      pltpu.async_copy(table_ref.at[idx_v.at[pl.ds((j0 + _AHEAD) * _G, _G)]],
                       rows_v.at[nslot], sem_g)

    z4 = (zeros16,) * _Q
    for v in range(1, _BLK):
      j1 = j0 + v
      slot1 = lax.rem(j1, _NBUF)

      @pl.when(j1 + _AHEAD < _NGROUPS)
      def _fire1(j1=j1):
        nslot = lax.rem(j1 + _AHEAD, _NBUF)
        pltpu.async_copy(
            table_ref.at[idx_v.at[pl.ds((j1 + _AHEAD) * _G, _G)]],
            rows_v.at[nslot], sem_g)

      _gwait(j1, slot1)
      p0 = j1 * _G
      b0 = (p0 * _DIV200_MUL) >> _DIV200_SHIFT
      bnd = jnp.minimum((b0 + 1) * _CUTLEN - p0, _G)

      def _accum(r, acc, slot1=slot1):
        return tuple(
            acc[q] + rows_v[slot1, r, pl.ds(q * 16, 16)] for q in range(_Q))

      acc_a = lax.fori_loop(0, bnd, _accum, z4)
      acc_b = lax.fori_loop(bnd, _G, _accum, z4)
      for q in range(_Q):
        acc_v[b0, pl.ds(q * 16, 16)] = (
            acc_v[b0, pl.ds(q * 16, 16)] + acc_a[q])

      @pl.when(bnd < _G)
      def _flush_b(b0=b0, acc_b=acc_b):
        for q in range(_Q):
          acc_v[b0 + 1, pl.ds(q * 16, 16)] = (
              acc_v[b0 + 1, pl.ds(q * 16, 16)] + acc_b[q])

    return carry

  lax.fori_loop(0, _NBLK, _block, 0)

  pltpu.make_async_copy(rows_v.at[(_NGROUPS - _BLK) % _NBUF],
                        acc_sh.at[seg_v.at[_NBLK - 1]], sem_s).wait()
  pltpu.sync_copy(acc_sh.at[pl.ds(s * _BPW, _BPW)], rows_v.at[0])

  def _merge_row(r, carry):
    for q in range(_Q):
      acc_v[r, pl.ds(q * 16, 16)] = (
          acc_v[r, pl.ds(q * 16, 16)] + rows_v[0, r, pl.ds(q * 16, 16)])
    return carry

  lax.fori_loop(0, _BPW, _merge_row, 0)
  pltpu.sync_copy(acc_v, out_ref.at[pl.ds(wid * _BPW, _BPW)])


@jax.jit
def _segment_sums(text, table):
  mesh = plsc.VectorSubcoreMesh(core_axis_name="c", subcore_axis_name="s",
                                num_cores=_NC, num_subcores=_NS)
  fn = pl.kernel(
      _sc_body,
      out_type=jax.ShapeDtypeStruct((_BATCH, _EMBED), jnp.float32),
      mesh=mesh,
      scratch_types=[
          pltpu.VMEM((_TPW,), jnp.int32),
          pltpu.VMEM((_NBLK, _G), jnp.int32),
          pltpu.VMEM((_NBUF, _G, _EMBED), jnp.float32),
          pltpu.VMEM((_BPW, _EMBED), jnp.float32),
          pltpu.VMEM_SHARED((_NS * _BPW, _EMBED), jnp.float32),
          pltpu.SemaphoreType.DMA,
          pltpu.SemaphoreType.DMA,
      ],
      compiler_params=pltpu.CompilerParams(use_tc_tiling_on_sc=False),
  )
  return fn(text, table)


def _tc_body(p_ref, w_ref, b_ref, o_ref):
  logits = jnp.dot(p_ref[...], w_ref[...],
                   preferred_element_type=jnp.float32) + b_ref[...]
  m = jnp.max(logits, axis=1, keepdims=True)
  e = jnp.exp(logits - m)
  probs = e / jnp.sum(e, axis=1, keepdims=True)
  o_ref[...] = probs.T


@jax.jit
def _dense_softmax(pooled_sum, wt, b2):
  out_t = pl.pallas_call(
      _tc_body,
      out_shape=jax.ShapeDtypeStruct((_NUM_CLASS, _BATCH), jnp.float32),
  )(pooled_sum, wt, b2)
  return out_t.T


def kernel(text, table, W, b):
  wt = (W.astype(jnp.float32) * (1.0 / _CUTLEN)).T
  b2 = b.reshape(1, _NUM_CLASS).astype(jnp.float32)

  pooled_sum = _segment_sums(text, table)
  return _dense_softmax(pooled_sum, wt, b2)


# speedup vs baseline: 1.3832x; 1.0008x over previous
import functools

import jax
import jax.numpy as jnp
import numpy as np
from jax import lax
from jax.experimental import pallas as pl
from jax.experimental.pallas import tpu as pltpu
from jax.experimental.pallas import tpu_sc as plsc

_VOCAB = 100000
_EMBED = 64
_NUM_CLASS = 4
_BATCH = 4096
_CUTLEN = 200

_NC = 2
_NS = 16
_NW = _NC * _NS
_BPW = _BATCH // _NW
_TPW = _BPW * _CUTLEN
_G = 128
_NGROUPS = _TPW // _G
_BLK = 4
_NVEC = _BLK - 1
_NBLK = _NGROUPS // _BLK

_NBUF = 10
_AHEAD = 6

_DIV200_MUL = 5243
_DIV200_SHIFT = 20

_Q = _EMBED // 16


def _sc_body(text_ref, table_ref, out_ref, idx_v, seg_v, rows_v, acc_v,
             acc_sh, sem_g, sem_s):
  c = lax.axis_index("c")
  s = lax.axis_index("s")
  wid = c * _NS + s

  pltpu.sync_copy(text_ref.at[pl.ds(wid * _TPW, _TPW)], idx_v)

  zeros16 = jnp.zeros((16,), jnp.float32)

  def _zero_row(r, carry):
    for q in range(_Q):
      rows_v[0, r, pl.ds(q * 16, 16)] = zeros16
      acc_v[r, pl.ds(q * 16, 16)] = zeros16
    return carry

  lax.fori_loop(0, _BPW, _zero_row, 0)
  pltpu.sync_copy(rows_v.at[0], acc_sh.at[pl.ds(s * _BPW, _BPW)])

  lanes = lax.iota(jnp.int32, 16)
  base = s * _BPW

  def _gen_seg(jj, carry):
    for q in range(_G // 16):
      p = lanes + (jj * _BLK * _G + q * 16)
      seg_v[jj, pl.ds(q * 16, 16)] = (
          base + ((p * _DIV200_MUL) >> _DIV200_SHIFT))
    return carry

  lax.fori_loop(0, _NBLK, _gen_seg, 0)

  for j in range(_AHEAD):
    pltpu.async_copy(table_ref.at[idx_v.at[pl.ds(j * _G, _G)]],
                     rows_v.at[j], sem_g)

  def _gwait(j, slot):
    pltpu.make_async_copy(table_ref.at[idx_v.at[pl.ds(j * _G, _G)]],
                          rows_v.at[slot], sem_g).wait()

  def _block(jj, carry):
    j0 = jj * _BLK
    slot0 = lax.rem(j0, _NBUF)

    _gwait(j0, slot0)
    pltpu.async_copy(rows_v.at[slot0], acc_sh.at[seg_v.at[jj]], sem_s,
                     add=True)

    @pl.when(jj >= 1)
    def _drain_prev():
      pslot = lax.rem(j0 - _BLK, _NBUF)
      pltpu.make_async_copy(rows_v.at[pslot], acc_sh.at[seg_v.at[jj - 1]],
                            sem_s).wait()

    @pl.when(j0 + _AHEAD < _NGROUPS)
    def _fire0():
      nslot = lax.rem(j0 + _AHEAD, _NBUF)
      pltpu.async_copy(table_ref.at[idx_v.at[pl.ds((j0 + _AHEAD) * _G, _G)]],
                       rows_v.at[nslot], sem_g)

    z4 = (zeros16,) * _Q
    for v in range(1, _BLK):
      j1 = j0 + v
      slot1 = lax.rem(j1, _NBUF)

      @pl.when(j1 + _AHEAD < _NGROUPS)
      def _fire1(j1=j1):
        nslot = lax.rem(j1 + _AHEAD, _NBUF)
        pltpu.async_copy(
            table_ref.at[idx_v.at[pl.ds((j1 + _AHEAD) * _G, _G)]],
            rows_v.at[nslot], sem_g)

      _gwait(j1, slot1)
      p0 = j1 * _G
      b0 = (p0 * _DIV200_MUL) >> _DIV200_SHIFT
      bnd = jnp.minimum((b0 + 1) * _CUTLEN - p0, _G)

      def _accum(r, acc, slot1=slot1):
        return tuple(
            acc[q] + rows_v[slot1, r, pl.ds(q * 16, 16)] for q in range(_Q))

      acc_a = plsc.parallel_loop(0, bnd, carry=z4, unroll=4)(_accum)
      acc_b = plsc.parallel_loop(bnd, _G, carry=z4, unroll=4)(_accum)
      for q in range(_Q):
        acc_v[b0, pl.ds(q * 16, 16)] = (
            acc_v[b0, pl.ds(q * 16, 16)] + acc_a[q])

      @pl.when(bnd < _G)
      def _flush_b(b0=b0, acc_b=acc_b):
        for q in range(_Q):
          acc_v[b0 + 1, pl.ds(q * 16, 16)] = (
              acc_v[b0 + 1, pl.ds(q * 16, 16)] + acc_b[q])

    return carry

  lax.fori_loop(0, _NBLK, _block, 0)

  pltpu.make_async_copy(rows_v.at[(_NGROUPS - _BLK) % _NBUF],
                        acc_sh.at[seg_v.at[_NBLK - 1]], sem_s).wait()
  pltpu.sync_copy(acc_sh.at[pl.ds(s * _BPW, _BPW)], rows_v.at[0])

  def _merge_row(r, carry):
    for q in range(_Q):
      acc_v[r, pl.ds(q * 16, 16)] = (
          acc_v[r, pl.ds(q * 16, 16)] + rows_v[0, r, pl.ds(q * 16, 16)])
    return carry

  lax.fori_loop(0, _BPW, _merge_row, 0)
  pltpu.sync_copy(acc_v, out_ref.at[pl.ds(wid * _BPW, _BPW)])


@jax.jit
def _segment_sums(text, table):
  mesh = plsc.VectorSubcoreMesh(core_axis_name="c", subcore_axis_name="s",
                                num_cores=_NC, num_subcores=_NS)
  fn = pl.kernel(
      _sc_body,
      out_type=jax.ShapeDtypeStruct((_BATCH, _EMBED), jnp.float32),
      mesh=mesh,
      scratch_types=[
          pltpu.VMEM((_TPW,), jnp.int32),
          pltpu.VMEM((_NBLK, _G), jnp.int32),
          pltpu.VMEM((_NBUF, _G, _EMBED), jnp.float32),
          pltpu.VMEM((_BPW, _EMBED), jnp.float32),
          pltpu.VMEM_SHARED((_NS * _BPW, _EMBED), jnp.float32),
          pltpu.SemaphoreType.DMA,
          pltpu.SemaphoreType.DMA,
      ],
      compiler_params=pltpu.CompilerParams(use_tc_tiling_on_sc=False),
  )
  return fn(text, table)


def _tc_body(p_ref, w_ref, b_ref, o_ref):
  logits = jnp.dot(p_ref[...], w_ref[...],
                   preferred_element_type=jnp.float32) + b_ref[...]
  m = jnp.max(logits, axis=1, keepdims=True)
  e = jnp.exp(logits - m)
  probs = e / jnp.sum(e, axis=1, keepdims=True)
  o_ref[...] = probs.T


@jax.jit
def _dense_softmax(pooled_sum, wt, b2):
  out_t = pl.pallas_call(
      _tc_body,
      out_shape=jax.ShapeDtypeStruct((_NUM_CLASS, _BATCH), jnp.float32),
  )(pooled_sum, wt, b2)
  return out_t.T


def kernel(text, table, W, b):
  wt = (W.astype(jnp.float32) * (1.0 / _CUTLEN)).T
  b2 = b.reshape(1, _NUM_CLASS).astype(jnp.float32)

  pooled_sum = _segment_sums(text, table)
  return _dense_softmax(pooled_sum, wt, b2)
